# baseline (speedup 1.0000x reference)
import functools

import jax
import jax.numpy as jnp
from jax.experimental import pallas as pl

NUM_EMBEDDINGS = 8192
EMBEDDING_DIM = 32
COMMITMENT_COST = 0.25

ROW_TILE = 128
CHUNK = 4096


def _vq_body(z_ref, z2_ref, wt_ref, w2_ref, w_ref, idx_ref, q_ref, loss_ref):
    zb = z_ref[...]
    mm = jnp.dot(zb.astype(jnp.bfloat16), wt_ref[...].astype(jnp.bfloat16),
                 preferred_element_type=jnp.float32)
    d = z2_ref[...] - 2.0 * mm + w2_ref[...]
    ii_c = jax.lax.broadcasted_iota(jnp.int32, (ROW_TILE, CHUNK), 1)
    acc = jnp.full((ROW_TILE, 1), jnp.inf, jnp.float32)
    sel_val = jnp.full((ROW_TILE, 1), jnp.inf, jnp.float32)
    sel_idx = jnp.zeros((ROW_TILE, 1), jnp.int32)
    for c in range(NUM_EMBEDDINGS // CHUNK):
        blk = d[:, c * CHUNK:(c + 1) * CHUNK]
        cm = jnp.min(blk, axis=1, keepdims=True)
        cidx = jnp.min(jnp.where(blk == cm, ii_c, jnp.int32(CHUNK)),
                       axis=1, keepdims=True) + c * CHUNK
        upd = cm < acc
        acc = jnp.where(upd, cm.astype(jnp.bfloat16).astype(jnp.float32), acc)
        sel_val = jnp.where(upd, cm, sel_val)
        sel_idx = jnp.where(upd, cidx, sel_idx)
    idx_ref[0, 0, :] = sel_idx[:, 0]
    dmin = sel_val
    ii = jax.lax.broadcasted_iota(jnp.int32, d.shape, 1)
    onehot = (ii == sel_idx).astype(jnp.float32)
    q = jnp.dot(onehot, w_ref[...], precision=jax.lax.Precision.HIGHEST,
                preferred_element_type=jnp.float32)
    q_ref[...] = zb + (q - zb)

    @pl.when(pl.program_id(0) == 0)
    def _init():
        loss_ref[...] = jnp.zeros((1, 1), jnp.float32)

    loss_ref[...] += jnp.sum(dmin).reshape(1, 1)


def kernel(z, W):
    D = W.shape[1]
    K = W.shape[0]
    flat_z = z.reshape(-1, D)
    N = flat_z.shape[0]
    n_tiles = N // ROW_TILE

    z2 = jnp.sum(flat_z ** 2, axis=1, keepdims=True)
    w2 = jnp.sum(W ** 2, axis=1).reshape(1, K)
    wt = W.T

    grid = (n_tiles,)
    idx_out, q_out, loss_sum = pl.pallas_call(
        _vq_body,
        grid=grid,
        in_specs=[
            pl.BlockSpec((ROW_TILE, D), lambda i: (i, 0)),
            pl.BlockSpec((ROW_TILE, 1), lambda i: (i, 0)),
            pl.BlockSpec((D, K), lambda i: (0, 0)),
            pl.BlockSpec((1, K), lambda i: (0, 0)),
            pl.BlockSpec((K, D), lambda i: (0, 0)),
        ],
        out_specs=[
            pl.BlockSpec((1, 1, ROW_TILE), lambda i: (i, 0, 0)),
            pl.BlockSpec((ROW_TILE, D), lambda i: (i, 0)),
            pl.BlockSpec((1, 1), lambda i: (0, 0)),
        ],
        out_shape=[
            jax.ShapeDtypeStruct((n_tiles, 1, ROW_TILE), jnp.int32),
            jax.ShapeDtypeStruct((N, D), jnp.float32),
            jax.ShapeDtypeStruct((1, 1), jnp.float32),
        ],
    )(flat_z, z2, wt, w2, W)

    quantized = q_out.reshape(z.shape)
    indices = idx_out.reshape(z.shape[0], -1)
    loss = loss_sum[0, 0] * ((1.0 + COMMITMENT_COST) / (N * D))
    return quantized, indices, loss


# speedup vs baseline: 2.6183x; 2.6183x over previous
import functools

import jax
import jax.numpy as jnp
from jax import lax
from jax.experimental import pallas as pl
from jax.experimental.pallas import tpu as pltpu
from jax.experimental.pallas import tpu_sc as plsc

NUM_EMBEDDINGS = 8192
EMBEDDING_DIM = 32
COMMITMENT_COST = 0.25

ROW_TILE = 128
CHUNK = 4096


def _vq_body(z_ref, z2_ref, wt_ref, w2_ref, idx_ref, loss_ref):
    zb = z_ref[...]
    mm = jnp.dot(zb.astype(jnp.bfloat16), wt_ref[...].astype(jnp.bfloat16),
                 preferred_element_type=jnp.float32)
    d = z2_ref[...] - 2.0 * mm + w2_ref[...]
    ii_c = jax.lax.broadcasted_iota(jnp.int32, (ROW_TILE, CHUNK), 1)
    acc = jnp.full((ROW_TILE, 1), jnp.inf, jnp.float32)
    sel_val = jnp.full((ROW_TILE, 1), jnp.inf, jnp.float32)
    sel_idx = jnp.zeros((ROW_TILE, 1), jnp.int32)
    for c in range(NUM_EMBEDDINGS // CHUNK):
        blk = d[:, c * CHUNK:(c + 1) * CHUNK]
        cm = jnp.min(blk, axis=1, keepdims=True)
        cidx = jnp.min(jnp.where(blk == cm, ii_c, jnp.int32(CHUNK)),
                       axis=1, keepdims=True) + c * CHUNK
        upd = cm < acc
        acc = jnp.where(upd, cm.astype(jnp.bfloat16).astype(jnp.float32), acc)
        sel_val = jnp.where(upd, cm, sel_val)
        sel_idx = jnp.where(upd, cidx, sel_idx)
    idx_ref[0, 0, :] = sel_idx[:, 0]

    @pl.when(pl.program_id(0) == 0)
    def _init():
        loss_ref[...] = jnp.zeros((1, 1), jnp.float32)

    loss_ref[...] += jnp.sum(sel_val).reshape(1, 1)


_SC_INFO = plsc.get_sparse_core_info()
_NW = _SC_INFO.num_cores * _SC_INFO.num_subcores
_GC = 128


def _sc_gather_body(table_hbm, idx_hbm, out_hbm, idx_v, rows_v, sem):
    wid = lax.axis_index("s") * _SC_INFO.num_cores + lax.axis_index("c")
    n_chunks = idx_v.shape[0]
    base = wid * (n_chunks * _GC)
    for j in range(n_chunks):
        off = base + j * _GC
        pltpu.sync_copy(idx_hbm.at[pl.ds(off, _GC)], idx_v.at[j])
        pltpu.async_copy(table_hbm.at[idx_v.at[j]], rows_v.at[j], sem).wait()
        pltpu.sync_copy(rows_v.at[j], out_hbm.at[pl.ds(off, _GC)])


def _sc_gather(W, idx_flat):
    B = idx_flat.shape[0]
    K = W.shape[0]
    W128 = jnp.pad(W, ((0, 0), (0, 128 - W.shape[1])))
    n_chunks = B // (_NW * _GC)
    k = functools.partial(
        pl.kernel,
        out_type=jax.ShapeDtypeStruct((B, 128), jnp.float32),
        mesh=plsc.VectorSubcoreMesh(core_axis_name="c", subcore_axis_name="s"),
        scratch_types=[
            pltpu.VMEM((n_chunks, _GC), jnp.int32),
            pltpu.VMEM((n_chunks, _GC, 128), jnp.float32),
            pltpu.SemaphoreType.DMA,
        ],
    )(_sc_gather_body)
    return k(W128, idx_flat)[:, :W.shape[1]]


def kernel(z, W):
    D = W.shape[1]
    K = W.shape[0]
    flat_z = z.reshape(-1, D)
    N = flat_z.shape[0]
    n_tiles = N // ROW_TILE

    z2 = jnp.sum(flat_z ** 2, axis=1, keepdims=True)
    w2 = jnp.sum(W ** 2, axis=1).reshape(1, K)
    wt = W.T

    idx_out, loss_sum = pl.pallas_call(
        _vq_body,
        grid=(n_tiles,),
        in_specs=[
            pl.BlockSpec((ROW_TILE, D), lambda i: (i, 0)),
            pl.BlockSpec((ROW_TILE, 1), lambda i: (i, 0)),
            pl.BlockSpec((D, K), lambda i: (0, 0)),
            pl.BlockSpec((1, K), lambda i: (0, 0)),
        ],
        out_specs=[
            pl.BlockSpec((1, 1, ROW_TILE), lambda i: (i, 0, 0)),
            pl.BlockSpec((1, 1), lambda i: (0, 0)),
        ],
        out_shape=[
            jax.ShapeDtypeStruct((n_tiles, 1, ROW_TILE), jnp.int32),
            jax.ShapeDtypeStruct((1, 1), jnp.float32),
        ],
    )(flat_z, z2, wt, w2)

    idx_flat = idx_out.reshape(N)
    quantized = _sc_gather(W, idx_flat).reshape(z.shape)
    indices = idx_out.reshape(z.shape[0], -1)
    loss = loss_sum[0, 0] * ((1.0 + COMMITMENT_COST) / (N * D))
    return quantized, indices, loss


# speedup vs baseline: 2.7992x; 1.0691x over previous
import functools

import jax
import jax.numpy as jnp
from jax import lax
from jax.experimental import pallas as pl
from jax.experimental.pallas import tpu as pltpu
from jax.experimental.pallas import tpu_sc as plsc

NUM_EMBEDDINGS = 8192
EMBEDDING_DIM = 32
COMMITMENT_COST = 0.25

ROW_TILE = 256
CHUNK = 4096


def _vq_body(z_ref, z2_ref, wt_ref, w2_ref, idx_ref, loss_ref):
    zb = z_ref[...]
    mm = jnp.dot(zb.astype(jnp.bfloat16), wt_ref[...].astype(jnp.bfloat16),
                 preferred_element_type=jnp.float32)
    d = z2_ref[...] - 2.0 * mm + w2_ref[...]
    ii_c = jax.lax.broadcasted_iota(jnp.int32, (ROW_TILE, CHUNK), 1)
    acc = jnp.full((ROW_TILE, 1), jnp.inf, jnp.float32)
    sel_val = jnp.full((ROW_TILE, 1), jnp.inf, jnp.float32)
    sel_idx = jnp.zeros((ROW_TILE, 1), jnp.int32)
    for c in range(NUM_EMBEDDINGS // CHUNK):
        blk = d[:, c * CHUNK:(c + 1) * CHUNK]
        cm = jnp.min(blk, axis=1, keepdims=True)
        cidx = jnp.min(jnp.where(blk == cm, ii_c, jnp.int32(CHUNK)),
                       axis=1, keepdims=True) + c * CHUNK
        upd = cm < acc
        acc = jnp.where(upd, cm.astype(jnp.bfloat16).astype(jnp.float32), acc)
        sel_val = jnp.where(upd, cm, sel_val)
        sel_idx = jnp.where(upd, cidx, sel_idx)
    idx_ref[0, 0, :] = sel_idx[:, 0]

    @pl.when(pl.program_id(0) == 0)
    def _init():
        loss_ref[...] = jnp.zeros((1, 1), jnp.float32)

    loss_ref[...] += jnp.sum(sel_val).reshape(1, 1)


_SC_INFO = plsc.get_sparse_core_info()
_NW = _SC_INFO.num_cores * _SC_INFO.num_subcores
_GC = 128


def _sc_gather_body(table_hbm, idx_hbm, out_hbm, idx_v, rows_v, sem):
    wid = lax.axis_index("s") * _SC_INFO.num_cores + lax.axis_index("c")
    n_chunks = idx_v.shape[0]
    base = wid * (n_chunks * _GC)
    for j in range(n_chunks):
        off = base + j * _GC
        pltpu.sync_copy(idx_hbm.at[pl.ds(off, _GC)], idx_v.at[j])
        pltpu.async_copy(table_hbm.at[idx_v.at[j]], rows_v.at[j], sem).wait()
        pltpu.sync_copy(rows_v.at[j], out_hbm.at[pl.ds(off, _GC)])


def _sc_gather(W, idx_flat):
    B = idx_flat.shape[0]
    K = W.shape[0]
    W128 = jnp.pad(W, ((0, 0), (0, 128 - W.shape[1])))
    n_chunks = B // (_NW * _GC)
    k = functools.partial(
        pl.kernel,
        out_type=jax.ShapeDtypeStruct((B, 128), jnp.float32),
        mesh=plsc.VectorSubcoreMesh(core_axis_name="c", subcore_axis_name="s"),
        scratch_types=[
            pltpu.VMEM((n_chunks, _GC), jnp.int32),
            pltpu.VMEM((n_chunks, _GC, 128), jnp.float32),
            pltpu.SemaphoreType.DMA,
        ],
    )(_sc_gather_body)
    return k(W128, idx_flat)[:, :W.shape[1]]


def kernel(z, W):
    D = W.shape[1]
    K = W.shape[0]
    flat_z = z.reshape(-1, D)
    N = flat_z.shape[0]
    n_tiles = N // ROW_TILE

    z2 = jnp.sum(flat_z ** 2, axis=1, keepdims=True)
    w2 = jnp.sum(W ** 2, axis=1).reshape(1, K)
    wt = W.T

    idx_out, loss_sum = pl.pallas_call(
        _vq_body,
        grid=(n_tiles,),
        in_specs=[
            pl.BlockSpec((ROW_TILE, D), lambda i: (i, 0)),
            pl.BlockSpec((ROW_TILE, 1), lambda i: (i, 0)),
            pl.BlockSpec((D, K), lambda i: (0, 0)),
            pl.BlockSpec((1, K), lambda i: (0, 0)),
        ],
        out_specs=[
            pl.BlockSpec((1, 1, ROW_TILE), lambda i: (i, 0, 0)),
            pl.BlockSpec((1, 1), lambda i: (0, 0)),
        ],
        out_shape=[
            jax.ShapeDtypeStruct((n_tiles, 1, ROW_TILE), jnp.int32),
            jax.ShapeDtypeStruct((1, 1), jnp.float32),
        ],
    )(flat_z, z2, wt, w2)

    idx_flat = idx_out.reshape(N)
    quantized = _sc_gather(W, idx_flat).reshape(z.shape)
    indices = idx_out.reshape(z.shape[0], -1)
    loss = loss_sum[0, 0] * ((1.0 + COMMITMENT_COST) / (N * D))
    return quantized, indices, loss
